# baseline (speedup 1.0000x reference)
import functools

import jax
import jax.numpy as jnp
from jax import lax
from jax.experimental import pallas as pl
from jax.experimental.pallas import tpu as pltpu
from jax.experimental.pallas import tpu_sc as plsc

BATCH = 16384
MEMBER_D = 32
SMALL_D = 16
OUT_D = MEMBER_D + 2 * SMALL_D
SMALL_V = 1000

NUM_CORES = 2
NUM_SUBCORES = 16
NUM_WORKERS = NUM_CORES * NUM_SUBCORES
BPW = BATCH // NUM_WORKERS
NPASS = 8
PB = BPW // NPASS
GRP = 16


def _mesh():
    return plsc.VectorSubcoreMesh(core_axis_name="c", subcore_axis_name="s")


def _splat(c):
    return jnp.full((GRP,), c, jnp.int32)


@functools.partial(
    pl.kernel,
    mesh=_mesh(),
    out_type=jax.ShapeDtypeStruct((OUT_D, BATCH), jnp.float32),
    compiler_params=pltpu.CompilerParams(needs_layout_passes=False),
    scratch_types=[
        pltpu.VMEM((NPASS, PB), jnp.int32),
        pltpu.VMEM((NPASS, PB), jnp.int32),
        pltpu.VMEM((NPASS, PB), jnp.int32),
        pltpu.VMEM((PB, 8, MEMBER_D), jnp.float32),
        pltpu.VMEM((SMALL_V // 8, 128), jnp.float32),
        pltpu.VMEM((SMALL_V // 8, 128), jnp.float32),
        pltpu.VMEM((OUT_D, 2 * PB), jnp.float32),
        pltpu.SemaphoreType.DMA,
    ],
)
def _encode(member_idx_hbm, party_idx_hbm, state_idx_hbm,
            member_tab_hbm, party_tab_hbm, state_tab_hbm, drain_hbm,
            out_hbm,
            midx_v, pidx_v, sidx_v, mblk, ptab, stab, orowsT, sem):
    wid = lax.axis_index("s") * NUM_CORES + lax.axis_index("c")
    base = wid * BPW

    pltpu.sync_copy(member_idx_hbm.at[wid], midx_v)
    pltpu.sync_copy(party_idx_hbm.at[wid], pidx_v)
    pltpu.sync_copy(state_idx_hbm.at[wid], sidx_v)
    pltpu.sync_copy(party_tab_hbm, ptab)
    pltpu.sync_copy(state_tab_hbm, stab)

    iota = lax.iota(jnp.int32, GRP)
    drain = pltpu.make_async_copy(drain_hbm, mblk, sem)

    def one_pass(p, col_off):
        def fire(g, carry2):
            mi = midx_v[p, pl.ds(g * GRP, GRP)]
            for l in range(GRP):
                b = mi[l]
                blk = pl.multiple_of(lax.bitwise_and(b, jnp.int32(-8)), 8)
                pltpu.async_copy(
                    member_tab_hbm.at[pl.ds(blk, 8)],
                    mblk.at[g * GRP + l], sem)
            return carry2
        lax.fori_loop(0, PB // GRP, fire, 0)

        drain.wait()

        def grp(g, carry2):
            rv = iota + g * GRP
            dst = pl.ds(col_off + g * GRP, GRP)
            mi = midx_v[p, pl.ds(g * GRP, GRP)]
            sub = lax.bitwise_and(mi, _splat(7))
            for c in range(MEMBER_D):
                orowsT[c, dst] = plsc.load_gather(
                    mblk, [rv, sub, _splat(c)])
            pi = pidx_v[p, pl.ds(g * GRP, GRP)]
            prow = lax.shift_right_logical(pi, _splat(3))
            pcol = lax.shift_left(lax.bitwise_and(pi, _splat(7)), _splat(4))
            for c in range(SMALL_D):
                orowsT[MEMBER_D + c, dst] = plsc.load_gather(
                    ptab, [prow, pcol + _splat(c)])
            si = sidx_v[p, pl.ds(g * GRP, GRP)]
            srow = lax.shift_right_logical(si, _splat(3))
            scol = lax.shift_left(lax.bitwise_and(si, _splat(7)), _splat(4))
            for c in range(SMALL_D):
                orowsT[MEMBER_D + SMALL_D + c, dst] = plsc.load_gather(
                    stab, [srow, scol + _splat(c)])
            return carry2
        lax.fori_loop(0, PB // GRP, grp, 0)

    def pair_body(q, carry):
        one_pass(2 * q, 0)
        one_pass(2 * q + 1, PB)
        off = pl.multiple_of(base + q * 2 * PB, 128)
        pltpu.sync_copy(orowsT, out_hbm.at[:, pl.ds(off, 2 * PB)])
        return carry

    lax.fori_loop(0, NPASS // 2, pair_body, 0)


def kernel(member, state, party, member_table, state_table, party_table):
    m = member.astype(jnp.int32).reshape(NUM_WORKERS, NPASS, PB)
    p = party.astype(jnp.int32).reshape(NUM_WORKERS, NPASS, PB)
    s = state.astype(jnp.int32).reshape(NUM_WORKERS, NPASS, PB)
    pt = party_table.reshape(SMALL_V // 8, 128)
    st = state_table.reshape(SMALL_V // 8, 128)
    dz = jnp.zeros((PB, 8, MEMBER_D), jnp.float32)
    return _encode(m, p, s, member_table, pt, st, dz).T


# speedup vs baseline: 1.0453x; 1.0453x over previous
import functools

import jax
import jax.numpy as jnp
from jax import lax
from jax.experimental import pallas as pl
from jax.experimental.pallas import tpu as pltpu
from jax.experimental.pallas import tpu_sc as plsc

BATCH = 16384
MEMBER_D = 32
SMALL_D = 16
OUT_D = MEMBER_D + 2 * SMALL_D
SMALL_V = 1000

NUM_CORES = 2
NUM_SUBCORES = 16
NUM_WORKERS = NUM_CORES * NUM_SUBCORES
BPW = BATCH // NUM_WORKERS
NPASS = 16
PB = BPW // NPASS
GRP = 16
QUAD = 4 * PB


def _mesh():
    return plsc.VectorSubcoreMesh(core_axis_name="c", subcore_axis_name="s")


def _splat(c):
    return jnp.full((GRP,), c, jnp.int32)


@functools.partial(
    pl.kernel,
    mesh=_mesh(),
    out_type=jax.ShapeDtypeStruct((OUT_D, BATCH), jnp.float32),
    compiler_params=pltpu.CompilerParams(needs_layout_passes=False),
    scratch_types=[
        pltpu.VMEM((NPASS, PB), jnp.int32),
        pltpu.VMEM((NPASS, PB), jnp.int32),
        pltpu.VMEM((NPASS, PB), jnp.int32),
        pltpu.VMEM((PB, 8, MEMBER_D), jnp.float32),
        pltpu.VMEM((PB, 8, MEMBER_D), jnp.float32),
        pltpu.VMEM((SMALL_V // 8, 128), jnp.float32),
        pltpu.VMEM((SMALL_V // 8, 128), jnp.float32),
        pltpu.VMEM((OUT_D, QUAD), jnp.float32),
        pltpu.SemaphoreType.DMA,
        pltpu.SemaphoreType.DMA,
    ],
)
def _encode(member_idx_hbm, party_idx_hbm, state_idx_hbm,
            member_tab_hbm, party_tab_hbm, state_tab_hbm, drain_hbm,
            out_hbm,
            midx_v, pidx_v, sidx_v, bufa, bufb, ptab, stab, orowsT,
            sema, semb):
    wid = lax.axis_index("s") * NUM_CORES + lax.axis_index("c")
    base = wid * BPW

    pltpu.sync_copy(member_idx_hbm.at[wid], midx_v)
    pltpu.sync_copy(party_idx_hbm.at[wid], pidx_v)
    pltpu.sync_copy(state_idx_hbm.at[wid], sidx_v)
    pltpu.sync_copy(party_tab_hbm, ptab)
    pltpu.sync_copy(state_tab_hbm, stab)

    iota = lax.iota(jnp.int32, GRP)
    draina = pltpu.make_async_copy(drain_hbm, bufa, sema)
    drainb = pltpu.make_async_copy(drain_hbm, bufb, semb)

    def fire(p, buf, sem):
        def fire_g(g, carry2):
            mi = midx_v[p, pl.ds(g * GRP, GRP)]
            for l in range(GRP):
                b = mi[l]
                blk = pl.multiple_of(lax.bitwise_and(b, jnp.int32(-8)), 8)
                pltpu.async_copy(
                    member_tab_hbm.at[pl.ds(blk, 8)],
                    buf.at[g * GRP + l], sem)
            return carry2
        lax.fori_loop(0, PB // GRP, fire_g, 0)

    def assemble(p, buf, col_off):
        def grp(g, carry2):
            rv = iota + g * GRP
            dst = pl.ds(col_off + g * GRP, GRP)
            mi = midx_v[p, pl.ds(g * GRP, GRP)]
            sub = lax.bitwise_and(mi, _splat(7))
            for c in range(MEMBER_D):
                orowsT[c, dst] = plsc.load_gather(
                    buf, [rv, sub, _splat(c)])
            pi = pidx_v[p, pl.ds(g * GRP, GRP)]
            prow = lax.shift_right_logical(pi, _splat(3))
            pcol = lax.shift_left(lax.bitwise_and(pi, _splat(7)), _splat(4))
            for c in range(SMALL_D):
                orowsT[MEMBER_D + c, dst] = plsc.load_gather(
                    ptab, [prow, pcol + _splat(c)])
            si = sidx_v[p, pl.ds(g * GRP, GRP)]
            srow = lax.shift_right_logical(si, _splat(3))
            scol = lax.shift_left(lax.bitwise_and(si, _splat(7)), _splat(4))
            for c in range(SMALL_D):
                orowsT[MEMBER_D + SMALL_D + c, dst] = plsc.load_gather(
                    stab, [srow, scol + _splat(c)])
            return carry2
        lax.fori_loop(0, PB // GRP, grp, 0)

    fire(0, bufa, sema)

    def quad_body(q, carry):
        p0 = 4 * q
        fire(p0 + 1, bufb, semb)
        draina.wait()
        assemble(p0, bufa, 0 * PB)
        fire(p0 + 2, bufa, sema)
        drainb.wait()
        assemble(p0 + 1, bufb, 1 * PB)
        fire(p0 + 3, bufb, semb)
        draina.wait()
        assemble(p0 + 2, bufa, 2 * PB)

        @pl.when(p0 + 4 < NPASS)
        def _():
            fire(p0 + 4, bufa, sema)
        drainb.wait()
        assemble(p0 + 3, bufb, 3 * PB)

        off = pl.multiple_of(base + q * QUAD, 128)
        pltpu.sync_copy(orowsT, out_hbm.at[:, pl.ds(off, QUAD)])
        return carry

    lax.fori_loop(0, NPASS // 4, quad_body, 0)


def kernel(member, state, party, member_table, state_table, party_table):
    m = member.astype(jnp.int32).reshape(NUM_WORKERS, NPASS, PB)
    p = party.astype(jnp.int32).reshape(NUM_WORKERS, NPASS, PB)
    s = state.astype(jnp.int32).reshape(NUM_WORKERS, NPASS, PB)
    pt = party_table.reshape(SMALL_V // 8, 128)
    st = state_table.reshape(SMALL_V // 8, 128)
    dz = jnp.zeros((PB, 8, MEMBER_D), jnp.float32)
    return _encode(m, p, s, member_table, pt, st, dz).T
